# baseline (speedup 1.0000x reference)
import jax
import jax.numpy as jnp
from jax import lax
from jax.experimental import pallas as pl
from jax.experimental.pallas import tpu as pltpu
from jax.experimental.pallas import tpu_sc as plsc

N_NODES = 10000
N_EDGES = 160000
DIM = 256
NC = 2
NS = 16
LANES = 16
CH = DIM // NC
EPT = N_EDGES // NS
CHUNK = 40
NSUP = 10
NSUB = 25
EGR = EPT // LANES
ZCH = 40
NZCH = N_NODES // ZCH


def _nrsqrt(x):
    i = plsc.bitcast(x, jnp.int32)
    i = jnp.int32(0x5F3759DF) - (i >> 1)
    y = plsc.bitcast(i, jnp.float32)
    for _ in range(3):
        y = y * (1.5 - 0.5 * x * y * y)
    return y


def _sc_dist_body(pos4, prows, pcols, out, postab, idxr, idxc, distbuf, sem):
    s = lax.axis_index("s")
    c = lax.axis_index("c")
    pltpu.sync_copy(pos4, postab)
    pltpu.sync_copy(prows.at[c, s], idxr)
    pltpu.sync_copy(pcols.at[c, s], idxc)

    def groupD(g, carry):
        rb = idxr[pl.ds(g * LANES, LANES)] * 4
        cb = idxc[pl.ds(g * LANES, LANES)] * 4

        def pcomp(base, comp):
            return plsc.load_gather(postab, [base + comp])

        dx = pcomp(rb, 0) - pcomp(cb, 0)
        dy = pcomp(rb, 1) - pcomp(cb, 1)
        dz = pcomp(rb, 2) - pcomp(cb, 2)
        d2 = jnp.maximum(dx * dx + dy * dy + dz * dz, 1e-24)
        distbuf[pl.ds(g * LANES, LANES)] = d2 * _nrsqrt(d2)
        return carry

    lax.fori_loop(0, EGR, groupD, None, unroll=False)
    pltpu.sync_copy(distbuf, out.at[c, s])


def _sc_dist(pos4, prows, pcols):
    mesh = plsc.VectorSubcoreMesh(core_axis_name="c", subcore_axis_name="s",
                                  num_cores=NC, num_subcores=NS)
    fn = pl.kernel(
        _sc_dist_body,
        out_type=jax.ShapeDtypeStruct((2, NS, EPT), jnp.float32),
        mesh=mesh,
        scratch_types=[
            pltpu.VMEM((4 * N_NODES,), jnp.float32),
            pltpu.VMEM((EPT,), jnp.int32),
            pltpu.VMEM((EPT,), jnp.int32),
            pltpu.VMEM((EPT,), jnp.float32),
            pltpu.SemaphoreType.DMA,
        ],
        compiler_params=pltpu.CompilerParams(needs_layout_passes=False),
    )
    return fn(pos4, prows, pcols)


_RBLK = 1600


def _tc_radial_body(dist_ref, wc_ref, vec_ref, out_ref):
    d = jnp.broadcast_to(dist_ref[...], (_RBLK, 128))
    lane = lax.broadcasted_iota(jnp.int32, (_RBLK, 128), 1)
    t = d - lane.astype(jnp.float32) * 1.125
    rbf = jnp.where(lane < 9, jnp.exp(-(t * t)), 0.0)
    h = jnp.dot(rbf.astype(jnp.bfloat16), wc_ref[...],
                preferred_element_type=jnp.float32)
    h = h + vec_ref[0:1, :]
    m = jnp.mean(h, axis=-1, keepdims=True)
    v = jnp.mean(h * h, axis=-1, keepdims=True) - m * m
    ln = (h - m) * lax.rsqrt(v + 1e-5) * vec_ref[1:2, :] + vec_ref[2:3, :]
    r = jnp.maximum(ln, 0.1 * ln).astype(jnp.bfloat16)
    out_ref[0, ...] = r[:, :CH]
    out_ref[1, ...] = r[:, CH:]


def _tc_radial(dist, wcp, vec):
    grid = (N_EDGES // _RBLK,)
    return pl.pallas_call(
        _tc_radial_body,
        grid=grid,
        in_specs=[pl.BlockSpec((_RBLK, 1), lambda i: (i, 0)),
                  pl.BlockSpec((128, DIM), lambda i: (0, 0)),
                  pl.BlockSpec((3, DIM), lambda i: (0, 0))],
        out_specs=pl.BlockSpec((NC, _RBLK, CH), lambda i: (0, i, 0)),
        out_shape=jax.ShapeDtypeStruct((NC, N_EDGES, CH), jnp.bfloat16),
    )(dist, wcp, vec)


def _sc_body(xcat, gidx, pcol4, rad, zblk, out,
             idxg, idxc, rb0, rb1, xb0, xb1, mb0, mb1, aggsh,
             sg0, sg1, sr0, sr1, ss0, ss1):
    c = lax.axis_index("c")
    s = lax.axis_index("s")
    slots = ((xb0, rb0, mb0, sg0, sr0, ss0), (xb1, rb1, mb1, sg1, sr1, ss1))

    for i in range((NZCH + NS - 1) // NS):
        zi = s + i * NS

        @pl.when(zi < NZCH)
        def _():
            pltpu.sync_copy(zblk, aggsh.at[pl.ds(zi * ZCH, ZCH)])
    plsc.subcore_barrier()

    def sup_body(sc, carry0):
        pltpu.sync_copy(gidx.at[c, s, sc], idxg)
        pltpu.sync_copy(pcol4.at[s, sc], idxc)
        e00 = s * EPT + sc * (NSUB * CHUNK)

        def issue(slot, j):
            xb, rb, mb, sg, sr, ss = slots[slot]
            pltpu.async_copy(xcat.at[idxg.at[j]], xb, sg)
            pltpu.async_copy(rad.at[c, pl.ds(e00 + j * CHUNK, CHUNK)],
                             rb, sr)

        def drain_scatter(slot, j):
            xb, rb, mb, sg, sr, ss = slots[slot]
            pltpu.make_async_copy(mb, aggsh.at[idxc.at[j]], ss).wait()

        def process(slot, j, nxt):
            xb, rb, mb, sg, sr, ss = slots[slot]
            pltpu.make_async_copy(xcat.at[idxg.at[j]], xb, sg).wait()
            pltpu.make_async_copy(
                rad.at[c, pl.ds(e00 + j * CHUNK, CHUNK)], rb, sr).wait()

            @pl.when(j >= 2)
            def _():
                drain_scatter(slot, j)

            def edge_mul(e, carry2):
                for t in range(CH // 32):
                    ra, rb_ = plsc.unpack(
                        rb[e, pl.ds(t * 32, 32)],
                        format=plsc.PackFormat.INTERLEAVED)
                    mb[e, pl.ds(t * 32, LANES)] = (
                        ra * xb[e, pl.ds(t * 32, LANES)])
                    mb[e, pl.ds(t * 32 + LANES, LANES)] = (
                        rb_ * xb[e, pl.ds(t * 32 + LANES, LANES)])
                return carry2

            lax.fori_loop(0, CHUNK, edge_mul, None, unroll=False)
            pltpu.async_copy(mb, aggsh.at[idxc.at[j]], ss, add=True)
            if nxt is not None:
                @pl.when(nxt < NSUB)
                def _():
                    issue(slot, nxt)

        issue(0, 0)
        issue(1, 1)

        def pair(kk, carry):
            j = kk * 2
            process(0, j, j + 2)
            process(1, j + 1, j + 3)
            return carry

        lax.fori_loop(0, NSUB // 2, pair, None, unroll=False)
        process(0, NSUB - 1, None)
        drain_scatter(1, NSUB - 2)
        drain_scatter(0, NSUB - 1)
        return carry0

    lax.fori_loop(0, NSUP, sup_body, None, unroll=False)
    plsc.subcore_barrier()

    for i in range((NZCH + NS - 1) // NS):
        zi = s + i * NS

        @pl.when(zi < NZCH)
        def _():
            pltpu.sync_copy(aggsh.at[pl.ds(zi * ZCH, ZCH)], mb0)
            pltpu.sync_copy(mb0, out.at[c, pl.ds(zi * ZCH, ZCH)])


_SC_SCRATCH = [
    pltpu.VMEM((NSUB, CHUNK), jnp.int32),
    pltpu.VMEM((NSUB, CHUNK), jnp.int32),
    pltpu.VMEM((CHUNK, CH), jnp.bfloat16),
    pltpu.VMEM((CHUNK, CH), jnp.bfloat16),
    pltpu.VMEM((CHUNK, CH), jnp.float32),
    pltpu.VMEM((CHUNK, CH), jnp.float32),
    pltpu.VMEM((CHUNK, CH), jnp.float32),
    pltpu.VMEM((CHUNK, CH), jnp.float32),
    pltpu.VMEM_SHARED((N_NODES, CH), jnp.float32),
    pltpu.SemaphoreType.DMA,
    pltpu.SemaphoreType.DMA,
    pltpu.SemaphoreType.DMA,
    pltpu.SemaphoreType.DMA,
    pltpu.SemaphoreType.DMA,
    pltpu.SemaphoreType.DMA,
]


def _sc_pass(xcat, gidx, pcol4, rad, zblk):
    mesh = plsc.VectorSubcoreMesh(core_axis_name="c", subcore_axis_name="s",
                                  num_cores=NC, num_subcores=NS)
    fn = pl.kernel(
        _sc_body,
        out_type=jax.ShapeDtypeStruct((NC, N_NODES, CH), jnp.float32),
        mesh=mesh,
        scratch_types=_SC_SCRATCH,
        compiler_params=pltpu.CompilerParams(needs_layout_passes=False),
    )
    return fn(xcat, gidx, pcol4, rad, zblk)


_BLK = 400


def _tc_body(alo_a, ahi_a, alo_b, ahi_b, x, Wo_a, Wo_b, vecs, out_ref):
    xb = x[...]

    def branch(alo, ahi, Wo, bo, go, betao):
        a = jnp.concatenate([alo[...], ahi[...]], axis=-1)
        h = jnp.dot(a, Wo[...], preferred_element_type=jnp.float32) + bo
        m = jnp.mean(h, axis=-1, keepdims=True)
        v = jnp.mean(h * h, axis=-1, keepdims=True) - m * m
        ln = (h - m) * lax.rsqrt(v + 1e-5) * go + betao
        return jnp.maximum(ln, 0.1 * ln)

    la = branch(alo_a, ahi_a, Wo_a, vecs[0:1, :], vecs[1:2, :], vecs[2:3, :])
    lb = branch(alo_b, ahi_b, Wo_b, vecs[3:4, :], vecs[4:5, :], vecs[5:6, :])
    out_ref[...] = 0.5 * (la + lb) + xb


def _tc_out(agg_a, agg_b, x, Wo_a, Wo_b, vecs):
    grid = (N_NODES // _BLK,)
    half_spec = pl.BlockSpec((_BLK, CH), lambda i: (i, 0))
    full_spec = pl.BlockSpec((_BLK, DIM), lambda i: (i, 0))
    w_spec = pl.BlockSpec((DIM, DIM), lambda i: (0, 0))
    v_spec = pl.BlockSpec((6, DIM), lambda i: (0, 0))
    return pl.pallas_call(
        _tc_body,
        grid=grid,
        in_specs=[half_spec, half_spec, half_spec, half_spec, full_spec,
                  w_spec, w_spec, v_spec],
        out_specs=full_spec,
        out_shape=jax.ShapeDtypeStruct((N_NODES, DIM), jnp.float32),
    )(agg_a[0], agg_a[1], agg_b[0], agg_b[1], x, Wo_a, Wo_b, vecs)


def kernel(x, pos, edge_index_intra, edge_index_inter, Wc_a, bc_a, gc_a, betac_a, Wo_a, bo_a, go_a, betao_a, Wc_b, bc_b, gc_b, betac_b, Wo_b, bo_b, go_b, betao_b):
    t = jnp.arange(CH)
    perm = (t // 32) * 32 + 2 * (t % 16) + (t % 32) // 16
    permf = jnp.concatenate([perm, CH + perm])

    xcat = jnp.concatenate([x[:, :CH][:, perm], x[:, CH:][:, perm]], axis=0)
    pos4 = jnp.pad(pos, ((0, 0), (0, 1))).reshape(-1)
    zblk = jnp.zeros((ZCH, CH), jnp.float32)

    rows = [edge_index_intra[0], edge_index_inter[0]]
    cols = [edge_index_intra[1], edge_index_inter[1]]
    prows = jnp.stack(rows).reshape(2, NS, EPT)
    pcols = jnp.stack(cols).reshape(2, NS, EPT)
    dist = _sc_dist(pos4, prows, pcols)

    def wprep(Wc, bc, gc, betac):
        wcp = jnp.zeros((128, DIM), jnp.bfloat16).at[:9, :].set(
            Wc.astype(jnp.bfloat16))
        vec = jnp.stack([bc, gc, betac])
        return wcp, vec

    def run_pass(p, Wc, bc, gc, betac):
        wcp, vec = wprep(Wc, bc, gc, betac)
        rad = _tc_radial(dist[p].reshape(N_EDGES, 1), wcp, vec)
        gidx = jnp.stack([rows[p], rows[p] + N_NODES]).reshape(
            NC, NS, NSUP, NSUB, CHUNK)
        pcol4 = cols[p].reshape(NS, NSUP, NSUB, CHUNK)
        return _sc_pass(xcat, gidx, pcol4, rad, zblk)

    agg_a = run_pass(0, Wc_a, bc_a, gc_a, betac_a)
    agg_b = run_pass(1, Wc_b, bc_b, gc_b, betac_b)

    vecs = jnp.stack([bo_a, go_a, betao_a, bo_b, go_b, betao_b])
    return _tc_out(agg_a, agg_b, x, Wo_a[permf, :], Wo_b[permf, :], vecs)


# speedup vs baseline: 1.5423x; 1.5423x over previous
import jax
import jax.numpy as jnp
from jax import lax
from jax.experimental import pallas as pl
from jax.experimental.pallas import tpu as pltpu
from jax.experimental.pallas import tpu_sc as plsc

N_NODES = 10000
N_EDGES = 160000
DIM = 256
NC = 2
NS = 16
LANES = 16
CH = DIM // NC
EPT = N_EDGES // NS
CHUNK = 80
NSUP = 5
NSUB = 25
EGR = EPT // LANES
ZCH = 80
NZCH = N_NODES // ZCH


def _nrsqrt(x):
    i = plsc.bitcast(x, jnp.int32)
    i = jnp.int32(0x5F3759DF) - (i >> 1)
    y = plsc.bitcast(i, jnp.float32)
    for _ in range(3):
        y = y * (1.5 - 0.5 * x * y * y)
    return y


def _sc_dist_body(pos4, prows, pcols, out, postab, idxr, idxc, distbuf, sem):
    s = lax.axis_index("s")
    c = lax.axis_index("c")
    pltpu.sync_copy(pos4, postab)
    pltpu.sync_copy(prows.at[c, s], idxr)
    pltpu.sync_copy(pcols.at[c, s], idxc)

    def groupD(g, carry):
        rb = idxr[pl.ds(g * LANES, LANES)] * 4
        cb = idxc[pl.ds(g * LANES, LANES)] * 4

        def pcomp(base, comp):
            return plsc.load_gather(postab, [base + comp])

        dx = pcomp(rb, 0) - pcomp(cb, 0)
        dy = pcomp(rb, 1) - pcomp(cb, 1)
        dz = pcomp(rb, 2) - pcomp(cb, 2)
        d2 = jnp.maximum(dx * dx + dy * dy + dz * dz, 1e-24)
        distbuf[pl.ds(g * LANES, LANES)] = d2 * _nrsqrt(d2)
        return carry

    lax.fori_loop(0, EGR, groupD, None, unroll=False)
    pltpu.sync_copy(distbuf, out.at[c, s])


def _sc_dist(pos4, prows, pcols):
    mesh = plsc.VectorSubcoreMesh(core_axis_name="c", subcore_axis_name="s",
                                  num_cores=NC, num_subcores=NS)
    fn = pl.kernel(
        _sc_dist_body,
        out_type=jax.ShapeDtypeStruct((2, NS, EPT), jnp.float32),
        mesh=mesh,
        scratch_types=[
            pltpu.VMEM((4 * N_NODES,), jnp.float32),
            pltpu.VMEM((EPT,), jnp.int32),
            pltpu.VMEM((EPT,), jnp.int32),
            pltpu.VMEM((EPT,), jnp.float32),
            pltpu.SemaphoreType.DMA,
        ],
        compiler_params=pltpu.CompilerParams(needs_layout_passes=False),
    )
    return fn(pos4, prows, pcols)


_RBLK = 1600


def _tc_radial_body(dist_ref, wc_ref, vec_ref, out_ref):
    d = jnp.broadcast_to(dist_ref[...], (_RBLK, 128))
    lane = lax.broadcasted_iota(jnp.int32, (_RBLK, 128), 1)
    t = d - lane.astype(jnp.float32) * 1.125
    rbf = jnp.where(lane < 9, jnp.exp(-(t * t)), 0.0)
    h = jnp.dot(rbf.astype(jnp.bfloat16), wc_ref[...],
                preferred_element_type=jnp.float32)
    h = h + vec_ref[0:1, :]
    m = jnp.mean(h, axis=-1, keepdims=True)
    v = jnp.mean(h * h, axis=-1, keepdims=True) - m * m
    ln = (h - m) * lax.rsqrt(v + 1e-5) * vec_ref[1:2, :] + vec_ref[2:3, :]
    r = jnp.maximum(ln, 0.1 * ln)
    out_ref[0, ...] = r[:, :CH]
    out_ref[1, ...] = r[:, CH:]


def _tc_radial(dist, wcp, vec):
    grid = (N_EDGES // _RBLK,)
    return pl.pallas_call(
        _tc_radial_body,
        grid=grid,
        in_specs=[pl.BlockSpec((_RBLK, 1), lambda i: (i, 0)),
                  pl.BlockSpec((128, DIM), lambda i: (0, 0)),
                  pl.BlockSpec((3, DIM), lambda i: (0, 0))],
        out_specs=pl.BlockSpec((NC, _RBLK, CH), lambda i: (0, i, 0)),
        out_shape=jax.ShapeDtypeStruct((NC, N_EDGES, CH), jnp.float32),
    )(dist, wcp, vec)


def _sc_body(xcat, gidx, pcol4, rad, zblk, out,
             idxg, idxc, rb0, rb1, xb0, xb1, aggsh,
             sg0, sg1, sr0, sr1):
    c = lax.axis_index("c")
    s = lax.axis_index("s")
    slots = ((xb0, rb0, sg0, sr0), (xb1, rb1, sg1, sr1))

    for i in range((NZCH + NS - 1) // NS):
        zi = s + i * NS

        @pl.when(zi < NZCH)
        def _():
            pltpu.sync_copy(zblk, aggsh.at[pl.ds(zi * ZCH, ZCH)])
    plsc.subcore_barrier()

    def sup_body(sc, carry0):
        pltpu.sync_copy(gidx.at[c, s, sc], idxg)
        pltpu.sync_copy(pcol4.at[s, sc], idxc)
        e00 = s * EPT + sc * (NSUB * CHUNK)

        def issue(slot, j):
            xb, rb, sg, sr = slots[slot]
            pltpu.async_copy(xcat.at[idxg.at[j]], xb, sg)
            pltpu.async_copy(rad.at[c, pl.ds(e00 + j * CHUNK, CHUNK)],
                             rb, sr)

        def process(slot, j, nxt):
            xb, rb, sg, sr = slots[slot]
            pltpu.make_async_copy(xcat.at[idxg.at[j]], xb, sg).wait()
            pltpu.make_async_copy(
                rad.at[c, pl.ds(e00 + j * CHUNK, CHUNK)], rb, sr).wait()

            def edge_mul(e, carry2):
                for t in range(CH // LANES):
                    sl = pl.ds(t * LANES, LANES)
                    xb[e, sl] = rb[e, sl] * xb[e, sl]
                return carry2

            lax.fori_loop(0, CHUNK, edge_mul, None, unroll=False)
            pltpu.sync_copy(xb, aggsh.at[idxc.at[j]], add=True)
            if nxt is not None:
                @pl.when(nxt < NSUB)
                def _():
                    issue(slot, nxt)

        issue(0, 0)
        issue(1, 1)

        def pair(kk, carry):
            j = kk * 2
            process(0, j, j + 2)
            process(1, j + 1, j + 3)
            return carry

        lax.fori_loop(0, NSUB // 2, pair, None, unroll=False)
        process(0, NSUB - 1, None)
        return carry0

    lax.fori_loop(0, NSUP, sup_body, None, unroll=False)
    plsc.subcore_barrier()

    for i in range((NZCH + NS - 1) // NS):
        zi = s + i * NS

        @pl.when(zi < NZCH)
        def _():
            pltpu.sync_copy(aggsh.at[pl.ds(zi * ZCH, ZCH)], xb0)
            pltpu.sync_copy(xb0, out.at[c, pl.ds(zi * ZCH, ZCH)])


_SC_SCRATCH = [
    pltpu.VMEM((NSUB, CHUNK), jnp.int32),
    pltpu.VMEM((NSUB, CHUNK), jnp.int32),
    pltpu.VMEM((CHUNK, CH), jnp.float32),
    pltpu.VMEM((CHUNK, CH), jnp.float32),
    pltpu.VMEM((CHUNK, CH), jnp.float32),
    pltpu.VMEM((CHUNK, CH), jnp.float32),
    pltpu.VMEM_SHARED((N_NODES, CH), jnp.float32),
    pltpu.SemaphoreType.DMA,
    pltpu.SemaphoreType.DMA,
    pltpu.SemaphoreType.DMA,
    pltpu.SemaphoreType.DMA,
]


def _sc_pass(xcat, gidx, pcol4, rad, zblk):
    mesh = plsc.VectorSubcoreMesh(core_axis_name="c", subcore_axis_name="s",
                                  num_cores=NC, num_subcores=NS)
    fn = pl.kernel(
        _sc_body,
        out_type=jax.ShapeDtypeStruct((NC, N_NODES, CH), jnp.float32),
        mesh=mesh,
        scratch_types=_SC_SCRATCH,
        compiler_params=pltpu.CompilerParams(needs_layout_passes=False),
    )
    return fn(xcat, gidx, pcol4, rad, zblk)


_BLK = 400


def _tc_body(alo_a, ahi_a, alo_b, ahi_b, x, Wo_a, Wo_b, vecs, out_ref):
    xb = x[...]

    def branch(alo, ahi, Wo, bo, go, betao):
        a = jnp.concatenate([alo[...], ahi[...]], axis=-1)
        h = jnp.dot(a, Wo[...], preferred_element_type=jnp.float32) + bo
        m = jnp.mean(h, axis=-1, keepdims=True)
        v = jnp.mean(h * h, axis=-1, keepdims=True) - m * m
        ln = (h - m) * lax.rsqrt(v + 1e-5) * go + betao
        return jnp.maximum(ln, 0.1 * ln)

    la = branch(alo_a, ahi_a, Wo_a, vecs[0:1, :], vecs[1:2, :], vecs[2:3, :])
    lb = branch(alo_b, ahi_b, Wo_b, vecs[3:4, :], vecs[4:5, :], vecs[5:6, :])
    out_ref[...] = 0.5 * (la + lb) + xb


def _tc_out(agg_a, agg_b, x, Wo_a, Wo_b, vecs):
    grid = (N_NODES // _BLK,)
    half_spec = pl.BlockSpec((_BLK, CH), lambda i: (i, 0))
    full_spec = pl.BlockSpec((_BLK, DIM), lambda i: (i, 0))
    w_spec = pl.BlockSpec((DIM, DIM), lambda i: (0, 0))
    v_spec = pl.BlockSpec((6, DIM), lambda i: (0, 0))
    return pl.pallas_call(
        _tc_body,
        grid=grid,
        in_specs=[half_spec, half_spec, half_spec, half_spec, full_spec,
                  w_spec, w_spec, v_spec],
        out_specs=full_spec,
        out_shape=jax.ShapeDtypeStruct((N_NODES, DIM), jnp.float32),
    )(agg_a[0], agg_a[1], agg_b[0], agg_b[1], x, Wo_a, Wo_b, vecs)


def kernel(x, pos, edge_index_intra, edge_index_inter, Wc_a, bc_a, gc_a, betac_a, Wo_a, bo_a, go_a, betao_a, Wc_b, bc_b, gc_b, betac_b, Wo_b, bo_b, go_b, betao_b):
    xcat = jnp.concatenate([x[:, :CH], x[:, CH:]], axis=0)
    pos4 = jnp.pad(pos, ((0, 0), (0, 1))).reshape(-1)
    zblk = jnp.zeros((ZCH, CH), jnp.float32)

    rows = [edge_index_intra[0], edge_index_inter[0]]
    cols = [edge_index_intra[1], edge_index_inter[1]]
    prows = jnp.stack(rows).reshape(2, NS, EPT)
    pcols = jnp.stack(cols).reshape(2, NS, EPT)
    dist = _sc_dist(pos4, prows, pcols)

    def wprep(Wc, bc, gc, betac):
        wcp = jnp.zeros((128, DIM), jnp.bfloat16).at[:9, :].set(
            Wc.astype(jnp.bfloat16))
        vec = jnp.stack([bc, gc, betac])
        return wcp, vec

    def run_pass(p, Wc, bc, gc, betac):
        wcp, vec = wprep(Wc, bc, gc, betac)
        rad = _tc_radial(dist[p].reshape(N_EDGES, 1), wcp, vec)
        gidx = jnp.stack([rows[p], rows[p] + N_NODES]).reshape(
            NC, NS, NSUP, NSUB, CHUNK)
        pcol4 = cols[p].reshape(NS, NSUP, NSUB, CHUNK)
        return _sc_pass(xcat, gidx, pcol4, rad, zblk)

    agg_a = run_pass(0, Wc_a, bc_a, gc_a, betac_a)
    agg_b = run_pass(1, Wc_b, bc_b, gc_b, betac_b)

    vecs = jnp.stack([bo_a, go_a, betao_a, bo_b, go_b, betao_b])
    return _tc_out(agg_a, agg_b, x, Wo_a, Wo_b, vecs)
